# Initial kernel scaffold; baseline (speedup 1.0000x reference)
import functools

import jax
import jax.numpy as jnp
from jax import lax
from jax.experimental import pallas as pl
from jax.experimental.pallas import tpu as pltpu
from jax.experimental.pallas import tpu_sc as plsc

B = 4096
L = 50
D = 32
HALF = 16

NC = 2
NS = 16
NW = NC * NS

TOTAL = 3 * B
S = 2
IDX = S * L
STEPS = TOTAL // (S * NW)
ROWS = STEPS * S
NBUF = 4


def _sc_pool_body(ids_hbm, table_hbm, out_hbm, idx_v, buf_v, out_v,
                  sem0, sem1, sem2, sem3):
    sems = (sem0, sem1, sem2, sem3)
    w = lax.axis_index("s") * NC + lax.axis_index("c")
    base = w * STEPS

    pltpu.sync_copy(ids_hbm.at[pl.ds(base, STEPS)], idx_v)

    def start(t, b):
        pltpu.make_async_copy(
            table_hbm.at[idx_v.at[t]], buf_v.at[b], sems[b]).start()

    for b in range(NBUF):
        start(b, b)

    def outer(g, carry):
        for b in range(NBUF):
            t = g * NBUF + b
            pltpu.make_async_copy(
                table_hbm.at[idx_v.at[t]], buf_v.at[b], sems[b]).wait()

            for r in range(S):
                def body(j, acc, _r=r, _b=b):
                    a0, a1 = acc
                    row = _r * L + j
                    a0 = a0 + buf_v[_b, row, pl.ds(0, HALF)]
                    a1 = a1 + buf_v[_b, row, pl.ds(HALF, HALF)]
                    return (a0, a1)

                z = jnp.zeros((HALF,), jnp.float32)
                a0, a1 = lax.fori_loop(0, L, body, (z, z))
                out_v[t * S + r, pl.ds(0, HALF)] = a0
                out_v[t * S + r, pl.ds(HALF, HALF)] = a1

            nxt = t + NBUF

            @pl.when(nxt < STEPS)
            def _(nxt=nxt, b=b):
                start(nxt, b)
        return carry

    lax.fori_loop(0, STEPS // NBUF, outer, 0)

    pltpu.sync_copy(out_v, out_hbm.at[pl.ds(w * ROWS, ROWS)])


_sc_pool = functools.partial(
    pl.kernel,
    out_type=jax.ShapeDtypeStruct((TOTAL, D), jnp.float32),
    mesh=plsc.VectorSubcoreMesh(core_axis_name="c", subcore_axis_name="s"),
    scratch_types=[
        pltpu.VMEM((STEPS, IDX), jnp.int32),
        pltpu.VMEM((NBUF, IDX, D), jnp.float32),
        pltpu.VMEM((ROWS, D), jnp.float32),
        pltpu.SemaphoreType.DMA,
        pltpu.SemaphoreType.DMA,
        pltpu.SemaphoreType.DMA,
        pltpu.SemaphoreType.DMA,
    ],
)(_sc_pool_body)


def _tc_loss_body(sums_ref, anchor_ref, loss_ref):
    a = sums_ref[0]
    p = sums_ref[1]
    n = sums_ref[2]
    an = a * lax.rsqrt(jnp.sum(a * a, axis=1, keepdims=True))
    pn = p * lax.rsqrt(jnp.sum(p * p, axis=1, keepdims=True))
    nn = n * lax.rsqrt(jnp.sum(n * n, axis=1, keepdims=True))
    anchor_ref[...] = an
    d_pos = jnp.sum((an - pn) ** 2, axis=1)
    d_neg = jnp.sum((an - nn) ** 2, axis=1)
    loss_ref[0, 0] = jnp.mean(jnp.maximum(1.0 + d_pos - d_neg, 0.0))


_tc_loss = pl.pallas_call(
    _tc_loss_body,
    out_shape=(
        jax.ShapeDtypeStruct((B, D), jnp.float32),
        jax.ShapeDtypeStruct((1, 1), jnp.float32),
    ),
)


def kernel(anchor_input_ids, positive_input_ids, negative_input_ids,
           embedding_table):
    ids = jnp.concatenate(
        [anchor_input_ids, positive_input_ids, negative_input_ids[0]], axis=0)
    ids2 = ids.reshape(TOTAL // S, IDX)
    pooled = _sc_pool(ids2, embedding_table)
    anchor, loss = _tc_loss(pooled.reshape(3, B, D))
    return anchor, loss[0, 0]


# speedup vs baseline: 3.7493x; 3.7493x over previous
import functools

import jax
import jax.numpy as jnp
from jax import lax
from jax.experimental import pallas as pl
from jax.experimental.pallas import tpu as pltpu
from jax.experimental.pallas import tpu_sc as plsc

B = 4096
L = 50
D = 32
HALF = 16

NC = 2
NS = 16
NW = NC * NS

TOTAL = 3 * B
S = 2
IDX = S * L
STEPS = TOTAL // (S * NW)
ROWS = STEPS * S
NBUF = 4


def _sc_pool_body(ids_hbm, table_hbm, out_hbm, idx_v, buf_v, out_v,
                  sem0, sem1, sem2, sem3):
    sems = (sem0, sem1, sem2, sem3)
    w = lax.axis_index("s") * NC + lax.axis_index("c")
    base = w * STEPS

    pltpu.sync_copy(ids_hbm.at[pl.ds(base, STEPS)], idx_v)

    def start(t, b):
        pltpu.make_async_copy(
            table_hbm.at[idx_v.at[t]], buf_v.at[b], sems[b]).start()

    for b in range(NBUF):
        start(b, b)

    def outer(g, carry):
        for b in range(NBUF):
            t = g * NBUF + b
            pltpu.make_async_copy(
                table_hbm.at[idx_v.at[t]], buf_v.at[b], sems[b]).wait()

            for r in range(S):
                def body(j, acc, _r=r, _b=b):
                    a0, a1 = acc
                    row = _r * L + j
                    a0 = a0 + buf_v[_b, row, pl.ds(0, HALF)]
                    a1 = a1 + buf_v[_b, row, pl.ds(HALF, HALF)]
                    return (a0, a1)

                z = jnp.zeros((HALF,), jnp.float32)
                a0, a1 = lax.fori_loop(0, L, body, (z, z))
                out_v[t * S + r, pl.ds(0, HALF)] = a0
                out_v[t * S + r, pl.ds(HALF, HALF)] = a1

            nxt = t + NBUF

            @pl.when(nxt < STEPS)
            def _(nxt=nxt, b=b):
                start(nxt, b)
        return carry

    lax.fori_loop(0, STEPS // NBUF, outer, 0)

    pltpu.sync_copy(out_v, out_hbm.at[pl.ds(w * ROWS, ROWS)])


_sc_pool = functools.partial(
    pl.kernel,
    out_type=jax.ShapeDtypeStruct((TOTAL, D), jnp.float32),
    mesh=plsc.VectorSubcoreMesh(core_axis_name="c", subcore_axis_name="s"),
    compiler_params=pltpu.CompilerParams(use_tc_tiling_on_sc=False),
    scratch_types=[
        pltpu.VMEM((STEPS, IDX), jnp.int32),
        pltpu.VMEM((NBUF, IDX, D), jnp.float32),
        pltpu.VMEM((ROWS, D), jnp.float32),
        pltpu.SemaphoreType.DMA,
        pltpu.SemaphoreType.DMA,
        pltpu.SemaphoreType.DMA,
        pltpu.SemaphoreType.DMA,
    ],
)(_sc_pool_body)


def _tc_loss_body(sums_ref, anchor_ref, loss_ref):
    a = sums_ref[0]
    p = sums_ref[1]
    n = sums_ref[2]
    an = a * lax.rsqrt(jnp.sum(a * a, axis=1, keepdims=True))
    pn = p * lax.rsqrt(jnp.sum(p * p, axis=1, keepdims=True))
    nn = n * lax.rsqrt(jnp.sum(n * n, axis=1, keepdims=True))
    anchor_ref[...] = an
    d_pos = jnp.sum((an - pn) ** 2, axis=1)
    d_neg = jnp.sum((an - nn) ** 2, axis=1)
    loss = jnp.mean(jnp.maximum(1.0 + d_pos - d_neg, 0.0))
    loss_ref[...] = jnp.reshape(loss, (1, 1))


_tc_loss = pl.pallas_call(
    _tc_loss_body,
    out_shape=(
        jax.ShapeDtypeStruct((B, D), jnp.float32),
        jax.ShapeDtypeStruct((1, 1), jnp.float32),
    ),
)


def kernel(anchor_input_ids, positive_input_ids, negative_input_ids,
           embedding_table):
    ids = jnp.concatenate(
        [anchor_input_ids, positive_input_ids, negative_input_ids[0]], axis=0)
    ids2 = ids.reshape(TOTAL // S, IDX)
    pooled = _sc_pool(ids2, embedding_table)
    anchor, loss = _tc_loss(pooled.reshape(3, B, D))
    return anchor, loss[0, 0]


# speedup vs baseline: 3.8842x; 1.0360x over previous
import functools

import jax
import jax.numpy as jnp
from jax import lax
from jax.experimental import pallas as pl
from jax.experimental.pallas import tpu as pltpu
from jax.experimental.pallas import tpu_sc as plsc

B = 4096
L = 50
D = 32
HALF = 16

NC = 2
NS = 16
NW = NC * NS

TOTAL = 3 * B
S = 2
IDX = S * L
STEPS = TOTAL // (S * NW)
ROWS = STEPS * S
NBUF = 4


def _sc_pool_body(ids_hbm, table_hbm, out_hbm, idx_v, buf_v, out_v,
                  sem0, sem1, sem2, sem3):
    sems = (sem0, sem1, sem2, sem3)
    w = lax.axis_index("s") * NC + lax.axis_index("c")
    base = w * STEPS

    pltpu.sync_copy(ids_hbm.at[pl.ds(base, STEPS)], idx_v)

    def start(t, b):
        pltpu.make_async_copy(
            table_hbm.at[idx_v.at[t]], buf_v.at[b], sems[b]).start()

    for b in range(NBUF):
        start(b, b)

    def outer(g, carry):
        for b in range(NBUF):
            t = g * NBUF + b
            pltpu.make_async_copy(
                table_hbm.at[idx_v.at[t]], buf_v.at[b], sems[b]).wait()

            for r in range(S):
                base = r * L
                a0 = buf_v[b, base + 0, pl.ds(0, HALF)]
                a1 = buf_v[b, base + 0, pl.ds(HALF, HALF)]
                a2 = buf_v[b, base + 1, pl.ds(0, HALF)]
                a3 = buf_v[b, base + 1, pl.ds(HALF, HALF)]
                for j in range(2, L, 2):
                    a0 = a0 + buf_v[b, base + j, pl.ds(0, HALF)]
                    a1 = a1 + buf_v[b, base + j, pl.ds(HALF, HALF)]
                    a2 = a2 + buf_v[b, base + j + 1, pl.ds(0, HALF)]
                    a3 = a3 + buf_v[b, base + j + 1, pl.ds(HALF, HALF)]
                out_v[t * S + r, pl.ds(0, HALF)] = a0 + a2
                out_v[t * S + r, pl.ds(HALF, HALF)] = a1 + a3

            nxt = t + NBUF

            @pl.when(nxt < STEPS)
            def _(nxt=nxt, b=b):
                start(nxt, b)
        return carry

    lax.fori_loop(0, STEPS // NBUF, outer, 0)

    pltpu.sync_copy(out_v, out_hbm.at[pl.ds(w * ROWS, ROWS)])


_sc_pool = functools.partial(
    pl.kernel,
    out_type=jax.ShapeDtypeStruct((TOTAL, D), jnp.float32),
    mesh=plsc.VectorSubcoreMesh(core_axis_name="c", subcore_axis_name="s"),
    compiler_params=pltpu.CompilerParams(use_tc_tiling_on_sc=False),
    scratch_types=[
        pltpu.VMEM((STEPS, IDX), jnp.int32),
        pltpu.VMEM((NBUF, IDX, D), jnp.float32),
        pltpu.VMEM((ROWS, D), jnp.float32),
        pltpu.SemaphoreType.DMA,
        pltpu.SemaphoreType.DMA,
        pltpu.SemaphoreType.DMA,
        pltpu.SemaphoreType.DMA,
    ],
)(_sc_pool_body)


def _tc_loss_body(sums_ref, anchor_ref, loss_ref):
    a = sums_ref[0]
    p = sums_ref[1]
    n = sums_ref[2]
    an = a * lax.rsqrt(jnp.sum(a * a, axis=1, keepdims=True))
    pn = p * lax.rsqrt(jnp.sum(p * p, axis=1, keepdims=True))
    nn = n * lax.rsqrt(jnp.sum(n * n, axis=1, keepdims=True))
    anchor_ref[...] = an
    d_pos = jnp.sum((an - pn) ** 2, axis=1)
    d_neg = jnp.sum((an - nn) ** 2, axis=1)
    loss = jnp.mean(jnp.maximum(1.0 + d_pos - d_neg, 0.0))
    loss_ref[...] = jnp.reshape(loss, (1, 1))


_tc_loss = pl.pallas_call(
    _tc_loss_body,
    out_shape=(
        jax.ShapeDtypeStruct((B, D), jnp.float32),
        jax.ShapeDtypeStruct((1, 1), jnp.float32),
    ),
)


def kernel(anchor_input_ids, positive_input_ids, negative_input_ids,
           embedding_table):
    ids = jnp.concatenate(
        [anchor_input_ids, positive_input_ids, negative_input_ids[0]], axis=0)
    ids2 = ids.reshape(TOTAL // S, IDX)
    pooled = _sc_pool(ids2, embedding_table)
    anchor, loss = _tc_loss(pooled.reshape(3, B, D))
    return anchor, loss[0, 0]


# speedup vs baseline: 3.9197x; 1.0092x over previous
import functools

import jax
import jax.numpy as jnp
from jax import lax
from jax.experimental import pallas as pl
from jax.experimental.pallas import tpu as pltpu
from jax.experimental.pallas import tpu_sc as plsc

B = 4096
L = 50
D = 32
HALF = 16

NC = 2
NS = 16
NW = NC * NS

TOTAL = 3 * B
S = 1
IDX = S * L
STEPS = TOTAL // (S * NW)
ROWS = STEPS * S
NBUF = 8


def _sc_pool_body(ids_hbm, table_hbm, out_hbm, idx_v, buf_v, out_v, *sems):
    w = lax.axis_index("s") * NC + lax.axis_index("c")
    base = w * STEPS

    pltpu.sync_copy(ids_hbm.at[pl.ds(base, STEPS)], idx_v)

    def start(t, b):
        pltpu.make_async_copy(
            table_hbm.at[idx_v.at[t]], buf_v.at[b], sems[b]).start()

    for b in range(NBUF):
        start(b, b)

    def outer(g, carry):
        for b in range(NBUF):
            t = g * NBUF + b
            pltpu.make_async_copy(
                table_hbm.at[idx_v.at[t]], buf_v.at[b], sems[b]).wait()

            for r in range(S):
                rb = r * L
                a0 = buf_v[b, rb + 0, pl.ds(0, HALF)]
                a1 = buf_v[b, rb + 0, pl.ds(HALF, HALF)]
                a2 = buf_v[b, rb + 1, pl.ds(0, HALF)]
                a3 = buf_v[b, rb + 1, pl.ds(HALF, HALF)]
                for j in range(2, L, 2):
                    a0 = a0 + buf_v[b, rb + j, pl.ds(0, HALF)]
                    a1 = a1 + buf_v[b, rb + j, pl.ds(HALF, HALF)]
                    a2 = a2 + buf_v[b, rb + j + 1, pl.ds(0, HALF)]
                    a3 = a3 + buf_v[b, rb + j + 1, pl.ds(HALF, HALF)]
                out_v[t * S + r, pl.ds(0, HALF)] = a0 + a2
                out_v[t * S + r, pl.ds(HALF, HALF)] = a1 + a3

            nxt = t + NBUF

            @pl.when(nxt < STEPS)
            def _(nxt=nxt, b=b):
                start(nxt, b)
        return carry

    lax.fori_loop(0, STEPS // NBUF, outer, 0)

    pltpu.sync_copy(out_v, out_hbm.at[pl.ds(w * ROWS, ROWS)])


_sc_pool = functools.partial(
    pl.kernel,
    out_type=jax.ShapeDtypeStruct((TOTAL, D), jnp.float32),
    mesh=plsc.VectorSubcoreMesh(core_axis_name="c", subcore_axis_name="s"),
    compiler_params=pltpu.CompilerParams(use_tc_tiling_on_sc=False),
    scratch_types=[
        pltpu.VMEM((STEPS, IDX), jnp.int32),
        pltpu.VMEM((NBUF, IDX, D), jnp.float32),
        pltpu.VMEM((ROWS, D), jnp.float32),
    ] + [pltpu.SemaphoreType.DMA] * NBUF,
)(_sc_pool_body)


def _tc_loss_body(sums_ref, anchor_ref, loss_ref):
    a = sums_ref[0]
    p = sums_ref[1]
    n = sums_ref[2]
    an = a * lax.rsqrt(jnp.sum(a * a, axis=1, keepdims=True))
    pn = p * lax.rsqrt(jnp.sum(p * p, axis=1, keepdims=True))
    nn = n * lax.rsqrt(jnp.sum(n * n, axis=1, keepdims=True))
    anchor_ref[...] = an
    d_pos = jnp.sum((an - pn) ** 2, axis=1)
    d_neg = jnp.sum((an - nn) ** 2, axis=1)
    loss = jnp.mean(jnp.maximum(1.0 + d_pos - d_neg, 0.0))
    loss_ref[...] = jnp.reshape(loss, (1, 1))


_tc_loss = pl.pallas_call(
    _tc_loss_body,
    out_shape=(
        jax.ShapeDtypeStruct((B, D), jnp.float32),
        jax.ShapeDtypeStruct((1, 1), jnp.float32),
    ),
)


def kernel(anchor_input_ids, positive_input_ids, negative_input_ids,
           embedding_table):
    ids = jnp.concatenate(
        [anchor_input_ids, positive_input_ids, negative_input_ids[0]], axis=0)
    pooled = _sc_pool(ids, embedding_table)
    anchor, loss = _tc_loss(pooled.reshape(3, B, D))
    return anchor, loss[0, 0]
